# baseline (speedup 1.0000x reference)
import functools

import jax
import jax.numpy as jnp
from jax import lax
from jax.experimental import pallas as pl
from jax.experimental.pallas import tpu as pltpu
from jax.experimental.pallas import tpu_sc as plsc

E = 320000
D = 128
R = 512

B = 5000

NW = 32
BPW = E // NW
C = 80
NBUF = 5
NGRP = BPW // (C * NBUF)
LANES = 16


def _sc_gather_body(table_hbm, idx_hbm, out_hbm, table_v, idx_v,
                    r0, r1, r2, r3, r4, wsems):
    rows = [r0, r1, r2, r3, r4]
    wid = lax.axis_index("s") * 2 + lax.axis_index("c")
    base = wid * BPW
    pltpu.sync_copy(table_hbm, table_v)
    pltpu.sync_copy(idx_hbm.at[pl.ds(base, BPW)], idx_v)
    lane_iota = lax.iota(jnp.int32, LANES)

    def fill(chunk, b):
        @pl.loop(0, C // LANES)
        def _g16(g):
            row_vec = idx_v[pl.ds(chunk * C + g * LANES, LANES)]
            row_off = row_vec * D
            e_off = (g * LANES + lane_iota) * D

            @pl.loop(0, D, unroll=8)
            def _col(j):
                val = plsc.load_gather(table_v, [row_off + j])
                plsc.store_scatter(rows[b], [e_off + j], val)

    def fire_write(chunk, b):
        pltpu.async_copy(
            rows[b], out_hbm.at[pl.ds((base + chunk * C) * D, C * D)],
            wsems.at[b])

    def wait_write(chunk, b):
        pltpu.make_async_copy(
            rows[b], out_hbm.at[pl.ds((base + chunk * C) * D, C * D)],
            wsems.at[b]).wait()

    for b in range(NBUF):
        fill(b, b)
        fire_write(b, b)

    @pl.loop(1, NGRP)
    def _group(k):
        c0 = k * NBUF
        for b in range(NBUF):
            wait_write(c0 - NBUF + b, b)
            fill(c0 + b, b)
            fire_write(c0 + b, b)

    for b in range(NBUF):
        wait_write((NGRP - 1) * NBUF + b, b)


def _sc_gather(table, idx):
    mesh = plsc.VectorSubcoreMesh(core_axis_name="c", subcore_axis_name="s")
    out_flat = pl.kernel(
        _sc_gather_body,
        out_type=jax.ShapeDtypeStruct((E * D,), jnp.float32),
        mesh=mesh,
        compiler_params=pltpu.CompilerParams(needs_layout_passes=False),
        scratch_types=[
            pltpu.VMEM((R * D,), jnp.float32),
            pltpu.VMEM((BPW,), jnp.int32),
            pltpu.VMEM((C * D,), jnp.float32),
            pltpu.VMEM((C * D,), jnp.float32),
            pltpu.VMEM((C * D,), jnp.float32),
            pltpu.VMEM((C * D,), jnp.float32),
            pltpu.VMEM((C * D,), jnp.float32),
            pltpu.SemaphoreType.DMA((NBUF,)),
        ],
    )(table.reshape(R * D), idx)
    return out_flat.reshape(E, D)


def _tc_score_kernel(idx_ref, table_ref, n1_ref, n2_ref, hn_ref, tn_ref,
                     pos_ref, nh_ref, nt_ref):
    idx = idx_ref[:, 0]
    iota = lax.broadcasted_iota(jnp.int32, (B, R), 1)
    onehot = (iota == idx[:, None]).astype(jnp.float32)
    r = jnp.dot(onehot, table_ref[...], preferred_element_type=jnp.float32)

    n1 = n1_ref[...]
    n2 = n2_ref[...]
    pos_ref[:, 0] = jnp.sum(jnp.abs(n1 * r - n2), axis=1)
    nh_ref[:, 0] = jnp.sum(jnp.abs(hn_ref[...] * r - n2), axis=1)
    nt_ref[:, 0] = jnp.sum(jnp.abs(n1 * r - tn_ref[...]), axis=1)


def _tc_scores(table, idx, n1, n2, hn, tn):
    idx2d = idx.reshape(E, 1)
    edge_spec = pl.BlockSpec((B, D), lambda i: (i, 0))
    score_spec = pl.BlockSpec((B, 1), lambda i: (i, 0))
    pos, nh, nt = pl.pallas_call(
        _tc_score_kernel,
        grid=(E // B,),
        in_specs=[
            pl.BlockSpec((B, 1), lambda i: (i, 0)),
            pl.BlockSpec((R, D), lambda i: (0, 0)),
            edge_spec, edge_spec, edge_spec, edge_spec,
        ],
        out_specs=[score_spec, score_spec, score_spec],
        out_shape=[
            jax.ShapeDtypeStruct((E, 1), jnp.float32),
            jax.ShapeDtypeStruct((E, 1), jnp.float32),
            jax.ShapeDtypeStruct((E, 1), jnp.float32),
        ],
    )(idx2d, table, n1, n2, hn, tn)
    return pos.reshape(E), nh.reshape(E), nt.reshape(E)


def kernel(update_rel_embed, edge_type, node1_encoder_result,
           node2_encoder_result, head_neg_encoder_result,
           tail_neg_encoder_result):
    idx = edge_type.astype(jnp.int32)
    r = _sc_gather(update_rel_embed, idx)
    pos, nh, nt = _tc_scores(update_rel_embed, idx, node1_encoder_result,
                             node2_encoder_result, head_neg_encoder_result,
                             tail_neg_encoder_result)
    return (pos, nh, nt, r)


# speedup vs baseline: 1.9910x; 1.9910x over previous
import functools

import jax
import jax.numpy as jnp
from jax import lax
from jax.experimental import pallas as pl
from jax.experimental.pallas import tpu as pltpu
from jax.experimental.pallas import tpu_sc as plsc

E = 320000
D = 128
R = 512

B = 5000

NW = 32
BPW = E // NW
C = 80
NBUF = 5
NGRP = BPW // (C * NBUF)
LANES = 16


def _sc_gather_body(table_hbm, idx_hbm, out_hbm, table_v, idx_v,
                    r0, r1, r2, r3, r4, wsems):
    rows = [r0, r1, r2, r3, r4]
    wid = lax.axis_index("s") * 2 + lax.axis_index("c")
    base = wid * BPW
    pltpu.sync_copy(table_hbm, table_v)
    pltpu.sync_copy(idx_hbm.at[pl.ds(base, BPW)], idx_v)
    lane_iota = lax.iota(jnp.int32, LANES)

    def fill(chunk, b):
        @plsc.parallel_loop(0, C // LANES)
        def _g16(g):
            row_vec = idx_v[pl.ds(chunk * C + g * LANES, LANES)]
            row_off = row_vec * D
            e_off = (g * LANES + lane_iota) * D

            @plsc.parallel_loop(0, D, unroll=8)
            def _col(j):
                val = plsc.load_gather(table_v, [row_off + j])
                plsc.store_scatter(rows[b], [e_off + j], val)

    def fire_write(chunk, b):
        pltpu.async_copy(
            rows[b], out_hbm.at[pl.ds((base + chunk * C) * D, C * D)],
            wsems.at[b])

    def wait_write(chunk, b):
        pltpu.make_async_copy(
            rows[b], out_hbm.at[pl.ds((base + chunk * C) * D, C * D)],
            wsems.at[b]).wait()

    for b in range(NBUF):
        fill(b, b)
        fire_write(b, b)

    @pl.loop(1, NGRP)
    def _group(k):
        c0 = k * NBUF
        for b in range(NBUF):
            wait_write(c0 - NBUF + b, b)
            fill(c0 + b, b)
            fire_write(c0 + b, b)

    for b in range(NBUF):
        wait_write((NGRP - 1) * NBUF + b, b)


def _sc_gather(table, idx):
    mesh = plsc.VectorSubcoreMesh(core_axis_name="c", subcore_axis_name="s")
    out_flat = pl.kernel(
        _sc_gather_body,
        out_type=jax.ShapeDtypeStruct((E * D,), jnp.float32),
        mesh=mesh,
        compiler_params=pltpu.CompilerParams(needs_layout_passes=False),
        scratch_types=[
            pltpu.VMEM((R * D,), jnp.float32),
            pltpu.VMEM((BPW,), jnp.int32),
            pltpu.VMEM((C * D,), jnp.float32),
            pltpu.VMEM((C * D,), jnp.float32),
            pltpu.VMEM((C * D,), jnp.float32),
            pltpu.VMEM((C * D,), jnp.float32),
            pltpu.VMEM((C * D,), jnp.float32),
            pltpu.SemaphoreType.DMA((NBUF,)),
        ],
    )(table.reshape(R * D), idx)
    return out_flat.reshape(E, D)


def _tc_score_kernel(idx_ref, table_ref, n1_ref, n2_ref, hn_ref, tn_ref,
                     pos_ref, nh_ref, nt_ref):
    idx = idx_ref[:, 0]
    iota = lax.broadcasted_iota(jnp.int32, (B, R), 1)
    onehot = (iota == idx[:, None]).astype(jnp.float32)
    r = jnp.dot(onehot, table_ref[...], preferred_element_type=jnp.float32)

    n1 = n1_ref[...]
    n2 = n2_ref[...]
    pos_ref[:, 0] = jnp.sum(jnp.abs(n1 * r - n2), axis=1)
    nh_ref[:, 0] = jnp.sum(jnp.abs(hn_ref[...] * r - n2), axis=1)
    nt_ref[:, 0] = jnp.sum(jnp.abs(n1 * r - tn_ref[...]), axis=1)


def _tc_scores(table, idx, n1, n2, hn, tn):
    idx2d = idx.reshape(E, 1)
    edge_spec = pl.BlockSpec((B, D), lambda i: (i, 0))
    score_spec = pl.BlockSpec((B, 1), lambda i: (i, 0))
    pos, nh, nt = pl.pallas_call(
        _tc_score_kernel,
        grid=(E // B,),
        in_specs=[
            pl.BlockSpec((B, 1), lambda i: (i, 0)),
            pl.BlockSpec((R, D), lambda i: (0, 0)),
            edge_spec, edge_spec, edge_spec, edge_spec,
        ],
        out_specs=[score_spec, score_spec, score_spec],
        out_shape=[
            jax.ShapeDtypeStruct((E, 1), jnp.float32),
            jax.ShapeDtypeStruct((E, 1), jnp.float32),
            jax.ShapeDtypeStruct((E, 1), jnp.float32),
        ],
    )(idx2d, table, n1, n2, hn, tn)
    return pos.reshape(E), nh.reshape(E), nt.reshape(E)


def kernel(update_rel_embed, edge_type, node1_encoder_result,
           node2_encoder_result, head_neg_encoder_result,
           tail_neg_encoder_result):
    idx = edge_type.astype(jnp.int32)
    r = _sc_gather(update_rel_embed, idx)
    pos, nh, nt = _tc_scores(update_rel_embed, idx, node1_encoder_result,
                             node2_encoder_result, head_neg_encoder_result,
                             tail_neg_encoder_result)
    return (pos, nh, nt, r)


# speedup vs baseline: 2.1213x; 1.0655x over previous
import functools

import jax
import jax.numpy as jnp
from jax import lax
from jax.experimental import pallas as pl
from jax.experimental.pallas import tpu as pltpu
from jax.experimental.pallas import tpu_sc as plsc

E = 320000
D = 128
R = 512

B = 5000

NW = 32
BPW = E // NW
C = 80
NBUF = 5
NGRP = BPW // (C * NBUF)
LANES = 16


def _sc_gather_body(table_hbm, idx_hbm, out_hbm, table_v, idx_v,
                    r0, r1, r2, r3, r4, wsems):
    rows = [r0, r1, r2, r3, r4]
    wid = lax.axis_index("s") * 2 + lax.axis_index("c")
    base = wid * BPW
    pltpu.sync_copy(table_hbm, table_v)
    pltpu.sync_copy(idx_hbm.at[pl.ds(base, BPW)], idx_v)
    lane_iota = lax.iota(jnp.int32, LANES)

    def fill(chunk, b):
        @plsc.parallel_loop(0, C, unroll=2)
        def _edge(e):
            row_splat = plsc.load_gather(
                idx_v, [jnp.full((LANES,), chunk * C + e, jnp.int32)])
            base = row_splat * D + lane_iota
            for jb in range(D // LANES):
                val = plsc.load_gather(table_v, [base + jb * LANES])
                rows[b][pl.ds(e * D + jb * LANES, LANES)] = val

    def fire_write(chunk, b):
        pltpu.async_copy(
            rows[b], out_hbm.at[pl.ds((base + chunk * C) * D, C * D)],
            wsems.at[b])

    def wait_write(chunk, b):
        pltpu.make_async_copy(
            rows[b], out_hbm.at[pl.ds((base + chunk * C) * D, C * D)],
            wsems.at[b]).wait()

    for b in range(NBUF):
        fill(b, b)
        fire_write(b, b)

    @pl.loop(1, NGRP)
    def _group(k):
        c0 = k * NBUF
        for b in range(NBUF):
            wait_write(c0 - NBUF + b, b)
            fill(c0 + b, b)
            fire_write(c0 + b, b)

    for b in range(NBUF):
        wait_write((NGRP - 1) * NBUF + b, b)


def _sc_gather(table, idx):
    mesh = plsc.VectorSubcoreMesh(core_axis_name="c", subcore_axis_name="s")
    out_flat = pl.kernel(
        _sc_gather_body,
        out_type=jax.ShapeDtypeStruct((E * D,), jnp.float32),
        mesh=mesh,
        compiler_params=pltpu.CompilerParams(needs_layout_passes=False),
        scratch_types=[
            pltpu.VMEM((R * D,), jnp.float32),
            pltpu.VMEM((BPW,), jnp.int32),
            pltpu.VMEM((C * D,), jnp.float32),
            pltpu.VMEM((C * D,), jnp.float32),
            pltpu.VMEM((C * D,), jnp.float32),
            pltpu.VMEM((C * D,), jnp.float32),
            pltpu.VMEM((C * D,), jnp.float32),
            pltpu.SemaphoreType.DMA((NBUF,)),
        ],
    )(table.reshape(R * D), idx)
    return out_flat.reshape(E, D)


def _tc_score_kernel(idx_ref, table_ref, n1_ref, n2_ref, hn_ref, tn_ref,
                     pos_ref, nh_ref, nt_ref):
    idx = idx_ref[:, 0]
    iota = lax.broadcasted_iota(jnp.int32, (B, R), 1)
    onehot = (iota == idx[:, None]).astype(jnp.float32)
    r = jnp.dot(onehot, table_ref[...], preferred_element_type=jnp.float32)

    n1 = n1_ref[...]
    n2 = n2_ref[...]
    pos_ref[:, 0] = jnp.sum(jnp.abs(n1 * r - n2), axis=1)
    nh_ref[:, 0] = jnp.sum(jnp.abs(hn_ref[...] * r - n2), axis=1)
    nt_ref[:, 0] = jnp.sum(jnp.abs(n1 * r - tn_ref[...]), axis=1)


def _tc_scores(table, idx, n1, n2, hn, tn):
    idx2d = idx.reshape(E, 1)
    edge_spec = pl.BlockSpec((B, D), lambda i: (i, 0))
    score_spec = pl.BlockSpec((B, 1), lambda i: (i, 0))
    pos, nh, nt = pl.pallas_call(
        _tc_score_kernel,
        grid=(E // B,),
        in_specs=[
            pl.BlockSpec((B, 1), lambda i: (i, 0)),
            pl.BlockSpec((R, D), lambda i: (0, 0)),
            edge_spec, edge_spec, edge_spec, edge_spec,
        ],
        out_specs=[score_spec, score_spec, score_spec],
        out_shape=[
            jax.ShapeDtypeStruct((E, 1), jnp.float32),
            jax.ShapeDtypeStruct((E, 1), jnp.float32),
            jax.ShapeDtypeStruct((E, 1), jnp.float32),
        ],
    )(idx2d, table, n1, n2, hn, tn)
    return pos.reshape(E), nh.reshape(E), nt.reshape(E)


def kernel(update_rel_embed, edge_type, node1_encoder_result,
           node2_encoder_result, head_neg_encoder_result,
           tail_neg_encoder_result):
    idx = edge_type.astype(jnp.int32)
    r = _sc_gather(update_rel_embed, idx)
    pos, nh, nt = _tc_scores(update_rel_embed, idx, node1_encoder_result,
                             node2_encoder_result, head_neg_encoder_result,
                             tail_neg_encoder_result)
    return (pos, nh, nt, r)
